# baseline (speedup 1.0000x reference)
import functools

import jax
import jax.numpy as jnp
from jax import lax
from jax.experimental import pallas as pl
from jax.experimental.pallas import tpu as pltpu
from jax.experimental.pallas import tpu_sc as plsc

MINUTE = 289
WEEK = 8
D = 64
L = 50
B = 16384
N = B * L
NW = 32
C = 128
FUSED = MINUTE * WEEK


def _fuse_tables(day, week):
    def body(day_ref, week_ref, out_ref):
        out_ref[...] = week_ref[...][:, None, :] + day_ref[...][None, :, :]

    return pl.pallas_call(
        body,
        out_shape=jax.ShapeDtypeStruct((WEEK, MINUTE, D), jnp.float32),
    )(day, week)


G = 8
BG = B // G
NG = BG * L


def _fuse_idx(xd, xw, g):
    RIN = 2048
    base = g * (BG // RIN)

    def body(xd_ref, xw_ref, out_ref):
        out_ref[...] = xw_ref[...] * MINUTE + xd_ref[...]

    return pl.pallas_call(
        body,
        grid=(BG // RIN,),
        in_specs=[
            pl.BlockSpec((RIN, L), lambda i: (i + base, 0)),
            pl.BlockSpec((RIN, L), lambda i: (i + base, 0)),
        ],
        out_specs=pl.BlockSpec((RIN, L), lambda i: (i, 0)),
        out_shape=jax.ShapeDtypeStruct((BG, L), jnp.int32),
    )(xd, xw)


PW = NG // NW
NCH = PW // C
NB = 5
SK = 2
NBLK = NCH // NB


def _sc_body(fused_hbm, idx_hbm, out_hbm, idxf, rows, *sems):
    gsems = sems[:NB]
    ssems = sems[NB:]
    wid = lax.axis_index("s") * 2 + lax.axis_index("c")
    rbase = wid * NCH
    obase = wid * PW

    pltpu.sync_copy(idx_hbm.at[pl.ds(rbase, NCH)], idxf)

    def gather_start(b, c):
        pltpu.async_copy(fused_hbm.at[idxf.at[c]], rows.at[b], gsems[b])

    def gather_wait(b, c):
        pltpu.make_async_copy(fused_hbm.at[idxf.at[c]], rows.at[b],
                              gsems[b]).wait()

    def scatter_start(b, c):
        pltpu.async_copy(rows.at[b],
                         out_hbm.at[pl.ds(obase + c * C, C), pl.ds(0, D)],
                         ssems[b])

    def scatter_wait(b):
        pltpu.make_async_copy(rows.at[b],
                              out_hbm.at[pl.ds(obase, C), pl.ds(0, D)],
                              ssems[b]).wait()

    def blk(k, carry):
        for t in range(NB):
            j = k * NB + t

            @pl.when(k >= 1)
            def _():
                scatter_wait(t)

            @pl.when(k < NBLK)
            def _():
                gather_start(t, j)

            bb = (t - SK) % NB
            c = j - SK
            guard = (k < NBLK) if t >= SK else (k >= 1)

            @pl.when(guard)
            def _():
                gather_wait(bb, c)
                scatter_start(bb, c)
        return carry

    lax.fori_loop(0, NBLK + 1, blk, 0)


def _sc_gather(fused, idx2):
    mesh = plsc.VectorSubcoreMesh(core_axis_name="c", subcore_axis_name="s")
    run = functools.partial(
        pl.kernel,
        mesh=mesh,
        compiler_params=pltpu.CompilerParams(use_tc_tiling_on_sc=False),
        out_type=jax.ShapeDtypeStruct((NG, 128), jnp.float32),
        scratch_types=[
            pltpu.VMEM((NCH, C), jnp.int32),
            pltpu.VMEM((NB, C, D), jnp.float32),
        ] + [pltpu.SemaphoreType.DMA] * (2 * NB),
    )(_sc_body)
    return run(fused, idx2)


def _to_blayout(x2, g, prev=None):
    BBB = 256
    base = g * (BG // BBB)

    def body(in_ref, *rest):
        out_ref = rest[-1]
        x = in_ref[...]
        x = x.reshape(BBB, L, 128)[:, :, :D]
        x = jnp.transpose(x, (1, 0, 2))
        out_ref[...] = jnp.swapaxes(x, 1, 2)

    in_specs = [pl.BlockSpec((BBB * L, 128), lambda i: (i, 0))]
    args = [x2]
    aliases = {}
    if prev is not None:
        in_specs.append(pl.BlockSpec(memory_space=pl.ANY))
        args.append(prev)
        aliases = {1: 0}

    return pl.pallas_call(
        body,
        grid=(BG // BBB,),
        in_specs=in_specs,
        out_specs=pl.BlockSpec((L, D, BBB), lambda i: (0, 0, i + base)),
        out_shape=jax.ShapeDtypeStruct((L, D, B), jnp.float32),
        input_output_aliases=aliases,
    )(*args)


@jax.jit
def kernel(x_day, x_week, daytime_table, weekday_table):
    fused = _fuse_tables(daytime_table, weekday_table).reshape(FUSED, D)
    out_t = None
    for g in range(G):
        idx2 = _fuse_idx(x_day, x_week, g).reshape(NG // C, C)
        out2 = _sc_gather(fused, idx2)
        out_t = _to_blayout(out2, g, out_t)
    return jnp.transpose(out_t, (2, 0, 1))


# speedup vs baseline: 1.0732x; 1.0732x over previous
import functools

import jax
import jax.numpy as jnp
from jax import lax
from jax.experimental import pallas as pl
from jax.experimental.pallas import tpu as pltpu
from jax.experimental.pallas import tpu_sc as plsc

MINUTE = 289
WEEK = 8
D = 64
L = 50
B = 16384
N = B * L
NW = 32
C = 128
FUSED = MINUTE * WEEK


def _fuse_tables(day, week):
    def body(day_ref, week_ref, out_ref):
        out_ref[...] = week_ref[...][:, None, :] + day_ref[...][None, :, :]

    return pl.pallas_call(
        body,
        out_shape=jax.ShapeDtypeStruct((WEEK, MINUTE, D), jnp.float32),
    )(day, week)


G = 4
BG = B // G
NG = BG * L


def _fuse_idx(xd, xw, g):
    RIN = 2048
    base = g * (BG // RIN)

    def body(xd_ref, xw_ref, out_ref):
        out_ref[...] = xw_ref[...] * MINUTE + xd_ref[...]

    return pl.pallas_call(
        body,
        grid=(BG // RIN,),
        in_specs=[
            pl.BlockSpec((RIN, L), lambda i: (i + base, 0)),
            pl.BlockSpec((RIN, L), lambda i: (i + base, 0)),
        ],
        out_specs=pl.BlockSpec((RIN, L), lambda i: (i, 0)),
        out_shape=jax.ShapeDtypeStruct((BG, L), jnp.int32),
    )(xd, xw)


PW = NG // NW
NCH = PW // C
NB = 5
SK = 2
NBLK = NCH // NB


def _sc_body(fused_hbm, idx_hbm, out_hbm, idxf, rows, *sems):
    gsems = sems[:NB]
    ssems = sems[NB:]
    wid = lax.axis_index("s") * 2 + lax.axis_index("c")
    rbase = wid * NCH
    obase = wid * PW

    pltpu.sync_copy(idx_hbm.at[pl.ds(rbase, NCH)], idxf)

    def gather_start(b, c):
        pltpu.async_copy(fused_hbm.at[idxf.at[c]], rows.at[b], gsems[b])

    def gather_wait(b, c):
        pltpu.make_async_copy(fused_hbm.at[idxf.at[c]], rows.at[b],
                              gsems[b]).wait()

    def scatter_start(b, c):
        pltpu.async_copy(rows.at[b],
                         out_hbm.at[pl.ds(obase + c * C, C), pl.ds(0, D)],
                         ssems[b])

    def scatter_wait(b):
        pltpu.make_async_copy(rows.at[b],
                              out_hbm.at[pl.ds(obase, C), pl.ds(0, D)],
                              ssems[b]).wait()

    def blk(k, carry):
        for t in range(NB):
            j = k * NB + t

            @pl.when(k >= 1)
            def _():
                scatter_wait(t)

            @pl.when(k < NBLK)
            def _():
                gather_start(t, j)

            bb = (t - SK) % NB
            c = j - SK
            guard = (k < NBLK) if t >= SK else (k >= 1)

            @pl.when(guard)
            def _():
                gather_wait(bb, c)
                scatter_start(bb, c)
        return carry

    lax.fori_loop(0, NBLK + 1, blk, 0)


def _sc_gather(fused, idx2):
    mesh = plsc.VectorSubcoreMesh(core_axis_name="c", subcore_axis_name="s")
    run = functools.partial(
        pl.kernel,
        mesh=mesh,
        compiler_params=pltpu.CompilerParams(use_tc_tiling_on_sc=False),
        out_type=jax.ShapeDtypeStruct((NG, 128), jnp.float32),
        scratch_types=[
            pltpu.VMEM((NCH, C), jnp.int32),
            pltpu.VMEM((NB, C, D), jnp.float32),
        ] + [pltpu.SemaphoreType.DMA] * (2 * NB),
    )(_sc_body)
    return run(fused, idx2)


def _to_blayout(x2, g, prev=None):
    BBB = 256
    base = g * (BG // BBB)

    def body(in_ref, *rest):
        out_ref = rest[-1]
        x = in_ref[...]
        x = x.reshape(BBB, L, 128)[:, :, :D]
        x = jnp.transpose(x, (1, 0, 2))
        out_ref[...] = jnp.swapaxes(x, 1, 2)

    in_specs = [pl.BlockSpec((BBB * L, 128), lambda i: (i, 0))]
    args = [x2]
    aliases = {}
    if prev is not None:
        in_specs.append(pl.BlockSpec(memory_space=pl.ANY))
        args.append(prev)
        aliases = {1: 0}

    return pl.pallas_call(
        body,
        grid=(BG // BBB,),
        in_specs=in_specs,
        out_specs=pl.BlockSpec((L, D, BBB), lambda i: (0, 0, i + base)),
        out_shape=jax.ShapeDtypeStruct((L, D, B), jnp.float32),
        input_output_aliases=aliases,
    )(*args)


@jax.jit
def kernel(x_day, x_week, daytime_table, weekday_table):
    fused = _fuse_tables(daytime_table, weekday_table).reshape(FUSED, D)
    out_t = None
    for g in range(G):
        idx2 = _fuse_idx(x_day, x_week, g).reshape(NG // C, C)
        out2 = _sc_gather(fused, idx2)
        out_t = _to_blayout(out2, g, out_t)
    return jnp.transpose(out_t, (2, 0, 1))
